# baseline (speedup 1.0000x reference)
import functools

import jax
import jax.numpy as jnp
from jax import lax
from jax.experimental import pallas as pl
from jax.experimental.pallas import tpu as pltpu
from jax.experimental.pallas import tpu_sc as plsc

_NC = 2
_NS = 16
_TILE_ROWS = 16


def kernel(x, pos_table):
    batch = x.shape[0]
    seq_len = x.shape[1]
    emb = pos_table.shape[1]
    flat = seq_len * emb
    tab = pos_table[:seq_len].reshape(flat)

    nw = _NC * _NS
    rows_per_w = batch // nw
    ncopies = rows_per_w // _TILE_ROWS

    mesh = plsc.VectorSubcoreMesh(
        core_axis_name="c", subcore_axis_name="s", num_cores=_NC
    )

    @functools.partial(
        pl.kernel,
        out_type=jax.ShapeDtypeStruct((batch, flat), pos_table.dtype),
        mesh=mesh,
        scratch_types=[
            pltpu.VMEM((_TILE_ROWS, flat), pos_table.dtype),
            pltpu.SemaphoreType.DMA,
        ],
    )
    def sc_broadcast(tab_hbm, out_hbm, tile_v, sem):
        wid = lax.axis_index("s") * _NC + lax.axis_index("c")
        base = wid * rows_per_w
        for r in range(_TILE_ROWS):
            pltpu.async_copy(tab_hbm, tile_v.at[r], sem)
        for r in range(_TILE_ROWS):
            pltpu.make_async_copy(tab_hbm, tile_v.at[r], sem).wait()
        for j in range(ncopies):
            pltpu.async_copy(
                tile_v,
                out_hbm.at[pl.ds(base + j * _TILE_ROWS, _TILE_ROWS), :],
                sem,
            )
        for j in range(ncopies):
            pltpu.make_async_copy(
                tile_v,
                out_hbm.at[pl.ds(base + j * _TILE_ROWS, _TILE_ROWS), :],
                sem,
            ).wait()

    out = sc_broadcast(tab)
    return out.reshape(batch, seq_len, emb)


# speedup vs baseline: 1.2799x; 1.2799x over previous
import jax
import jax.numpy as jnp
from jax.experimental import pallas as pl
from jax.experimental.pallas import tpu as pltpu

_BLOCK_B = 1024


def _bcast_kernel(tab_ref, out_ref):
    out_ref[...] = jnp.broadcast_to(tab_ref[...], out_ref.shape)


def kernel(x, pos_table):
    batch = x.shape[0]
    seq_len = x.shape[1]
    emb = pos_table.shape[1]
    flat = seq_len * emb
    tab = pos_table[:seq_len].reshape(1, flat)

    block_b = _BLOCK_B if batch % _BLOCK_B == 0 else batch
    grid = (batch // block_b,)

    out = pl.pallas_call(
        _bcast_kernel,
        grid=grid,
        in_specs=[pl.BlockSpec((1, flat), lambda i: (0, 0))],
        out_specs=pl.BlockSpec((block_b, flat), lambda i: (i, 0)),
        out_shape=jax.ShapeDtypeStruct((batch, flat), pos_table.dtype),
        compiler_params=pltpu.CompilerParams(
            dimension_semantics=("arbitrary",),
            vmem_limit_bytes=100 * 1024 * 1024,
        ),
    )(tab)
    return out.reshape(batch, seq_len, emb)
